# Initial kernel scaffold; baseline (speedup 1.0000x reference)
import jax
import jax.numpy as jnp
from jax.experimental import pallas as pl


def kernel(x, class_mask, key_pool, anchor_pool, W_head, b_head):
    raise NotImplementedError("write your pallas kernel here")


# speedup vs baseline: 3.6758x; 3.6758x over previous
import functools
import jax
import jax.numpy as jnp
from jax.experimental import pallas as pl
from jax.experimental.pallas import tpu as pltpu

KEY_SZ = 768
NCLS = 200
BATCH = 8192
BLK = 512

_DEF = jax.lax.Precision.DEFAULT
_HI = jax.lax.Precision.HIGHEST


def _body(x_ref, key_ref, anchor_ref, w1_ref, w2_ref, b_ref,
          out_ref, x1_ref, rsum_ref, keyn_ref, aw2_ref):
    i = pl.program_id(0)

    @pl.when(i == 0)
    def _init():
        kp = key_ref[...]
        ss = jnp.sum(kp * kp, axis=1, keepdims=True)
        keyn_ref[...] = kp * jax.lax.rsqrt(jnp.maximum(ss, 1e-12))
        aw2_ref[...] = jax.lax.dot_general(
            anchor_ref[...], w2_ref[...], (((1,), (0,)), ((), ())),
            precision=_DEF, preferred_element_type=jnp.float32)
        rsum_ref[0, 0] = 0.0

    x = x_ref[...]
    ss = jnp.sum(x * x, axis=1, keepdims=True)
    xn = x * jax.lax.rsqrt(jnp.maximum(ss, 1e-12))
    keyn = keyn_ref[...]

    sim = jax.lax.dot_general(
        xn, keyn, (((1,), (1,)), ((), ())),
        precision=_DEF, preferred_element_type=jnp.float32)

    m = jnp.max(sim, axis=1, keepdims=True)
    iota = jax.lax.broadcasted_iota(jnp.int32, (BLK, NCLS), 1)
    idx = jnp.min(jnp.where(sim == m, iota, NCLS), axis=1, keepdims=True)
    onehot = (iota == idx).astype(jnp.float32)

    anchor = jax.lax.dot_general(
        onehot, anchor_ref[...], (((1,), (0,)), ((), ())),
        precision=_HI, preferred_element_type=jnp.float32)
    keysel = jax.lax.dot_general(
        onehot, keyn, (((1,), (0,)), ((), ())),
        precision=_HI, preferred_element_type=jnp.float32)

    rsum_ref[0, 0] += jnp.sum(keysel * xn)

    out = (jax.lax.dot_general(
               x, w1_ref[...], (((1,), (0,)), ((), ())),
               precision=_DEF, preferred_element_type=jnp.float32)
           + jax.lax.dot_general(
               onehot, aw2_ref[...], (((1,), (0,)), ((), ())),
               precision=_HI, preferred_element_type=jnp.float32)
           + b_ref[...])
    out_ref[...] = out
    x1_ref[:, :KEY_SZ] = x
    x1_ref[:, KEY_SZ:] = anchor


@jax.jit
def _run(x, key_pool, anchor_pool, w1, w2, b2d):
    grid = BATCH // BLK
    out, x1, rsum = pl.pallas_call(
        _body,
        grid=(grid,),
        in_specs=[
            pl.BlockSpec((BLK, KEY_SZ), lambda i: (i, 0)),
            pl.BlockSpec((NCLS, KEY_SZ), lambda i: (0, 0)),
            pl.BlockSpec((NCLS, KEY_SZ), lambda i: (0, 0)),
            pl.BlockSpec((KEY_SZ, NCLS), lambda i: (0, 0)),
            pl.BlockSpec((KEY_SZ, NCLS), lambda i: (0, 0)),
            pl.BlockSpec((1, NCLS), lambda i: (0, 0)),
        ],
        out_specs=[
            pl.BlockSpec((BLK, NCLS), lambda i: (i, 0)),
            pl.BlockSpec((BLK, 2 * KEY_SZ), lambda i: (i, 0)),
            pl.BlockSpec(memory_space=pltpu.SMEM),
        ],
        out_shape=[
            jax.ShapeDtypeStruct((BATCH, NCLS), jnp.float32),
            jax.ShapeDtypeStruct((BATCH, 2 * KEY_SZ), jnp.float32),
            jax.ShapeDtypeStruct((1, 1), jnp.float32),
        ],
        scratch_shapes=[
            pltpu.VMEM((NCLS, KEY_SZ), jnp.float32),
            pltpu.VMEM((NCLS, NCLS), jnp.float32),
        ],
    )(x, key_pool, anchor_pool, w1, w2, b2d)
    return out, x1, rsum[0, 0] / KEY_SZ


def kernel(x, class_mask, key_pool, anchor_pool, W_head, b_head):
    w1 = W_head[:KEY_SZ]
    w2 = W_head[KEY_SZ:]
    b2d = b_head.reshape(1, NCLS)
    return _run(x, key_pool, anchor_pool, w1, w2, b2d)


# speedup vs baseline: 4.5093x; 1.2268x over previous
import functools
import jax
import jax.numpy as jnp
from jax.experimental import pallas as pl
from jax.experimental.pallas import tpu as pltpu

KEY_SZ = 768
NCLS = 200
BATCH = 8192
BLK = 512

_DEF = jax.lax.Precision.DEFAULT
_HI = jax.lax.Precision.HIGHEST


def _body(x_ref, key_ref, anchor_ref, w1_ref, w2_ref, b_ref,
          out_ref, x1_ref, rsum_ref, keyn_ref, aw2_ref):
    i = pl.program_id(0)

    @pl.when(i == 0)
    def _init():
        kp = key_ref[...]
        ss = jnp.sum(kp * kp, axis=1, keepdims=True)
        keyn_ref[...] = kp * jax.lax.rsqrt(jnp.maximum(ss, 1e-12))
        aw2_ref[...] = jax.lax.dot_general(
            anchor_ref[...], w2_ref[...], (((1,), (0,)), ((), ())),
            precision=_DEF, preferred_element_type=jnp.float32)
        rsum_ref[0, 0] = 0.0

    x = x_ref[...]
    ss = jnp.sum(x * x, axis=1, keepdims=True)
    xn = x * jax.lax.rsqrt(jnp.maximum(ss, 1e-12))
    keyn = keyn_ref[...]

    sim = jax.lax.dot_general(
        xn, keyn, (((1,), (1,)), ((), ())),
        precision=_DEF, preferred_element_type=jnp.float32)

    m = jnp.max(sim, axis=1, keepdims=True)
    iota = jax.lax.broadcasted_iota(jnp.int32, (BLK, NCLS), 1)
    idx = jnp.min(jnp.where(sim == m, iota, NCLS), axis=1, keepdims=True)
    onehot = (iota == idx).astype(jnp.float32)

    anchor = jax.lax.dot_general(
        onehot, anchor_ref[...], (((1,), (0,)), ((), ())),
        precision=_DEF, preferred_element_type=jnp.float32)
    keysel = jax.lax.dot_general(
        onehot, keyn, (((1,), (0,)), ((), ())),
        precision=_HI, preferred_element_type=jnp.float32)

    rsum_ref[0, 0] += jnp.sum(keysel * xn)

    out = (jax.lax.dot_general(
               x, w1_ref[...], (((1,), (0,)), ((), ())),
               precision=_DEF, preferred_element_type=jnp.float32)
           + jax.lax.dot_general(
               onehot, aw2_ref[...], (((1,), (0,)), ((), ())),
               precision=_DEF, preferred_element_type=jnp.float32)
           + b_ref[...])
    out_ref[...] = out
    x1_ref[:, :KEY_SZ] = x
    x1_ref[:, KEY_SZ:] = anchor


@jax.jit
def _run(x, key_pool, anchor_pool, w1, w2, b2d):
    grid = BATCH // BLK
    out, x1, rsum = pl.pallas_call(
        _body,
        grid=(grid,),
        in_specs=[
            pl.BlockSpec((BLK, KEY_SZ), lambda i: (i, 0)),
            pl.BlockSpec((NCLS, KEY_SZ), lambda i: (0, 0)),
            pl.BlockSpec((NCLS, KEY_SZ), lambda i: (0, 0)),
            pl.BlockSpec((KEY_SZ, NCLS), lambda i: (0, 0)),
            pl.BlockSpec((KEY_SZ, NCLS), lambda i: (0, 0)),
            pl.BlockSpec((1, NCLS), lambda i: (0, 0)),
        ],
        out_specs=[
            pl.BlockSpec((BLK, NCLS), lambda i: (i, 0)),
            pl.BlockSpec((BLK, 2 * KEY_SZ), lambda i: (i, 0)),
            pl.BlockSpec(memory_space=pltpu.SMEM),
        ],
        out_shape=[
            jax.ShapeDtypeStruct((BATCH, NCLS), jnp.float32),
            jax.ShapeDtypeStruct((BATCH, 2 * KEY_SZ), jnp.float32),
            jax.ShapeDtypeStruct((1, 1), jnp.float32),
        ],
        scratch_shapes=[
            pltpu.VMEM((NCLS, KEY_SZ), jnp.float32),
            pltpu.VMEM((NCLS, NCLS), jnp.float32),
        ],
    )(x, key_pool, anchor_pool, w1, w2, b2d)
    return out, x1, rsum[0, 0] / KEY_SZ


def kernel(x, class_mask, key_pool, anchor_pool, W_head, b_head):
    w1 = W_head[:KEY_SZ]
    w2 = W_head[KEY_SZ:]
    b2d = b_head.reshape(1, NCLS)
    return _run(x, key_pool, anchor_pool, w1, w2, b2d)


# speedup vs baseline: 5.8977x; 1.3079x over previous
import functools
import jax
import jax.numpy as jnp
from jax.experimental import pallas as pl
from jax.experimental.pallas import tpu as pltpu

KEY_SZ = 768
NCLS = 200
NPAD = 256
BATCH = 8192
BLK = 512
TCOLS = 3 * KEY_SZ + NCLS
TPADC = 2560

_DEF = jax.lax.Precision.DEFAULT


def _body(x_ref, key_ref, anchor_ref, w1_ref, w2_ref, b_ref,
          out_ref, x1_ref, rsum_ref, keyn_ref, tcat_ref, racc_ref):
    i = pl.program_id(0)

    @pl.when(i == 0)
    def _init():
        kp = key_ref[...]
        ss = jnp.sum(kp * kp, axis=1, keepdims=True)
        keyn = kp * jax.lax.rsqrt(jnp.maximum(ss, 1e-12))
        keyn_ref[...] = keyn
        aw2 = jax.lax.dot_general(
            anchor_ref[...], w2_ref[...], (((1,), (0,)), ((), ())),
            precision=_DEF, preferred_element_type=jnp.float32)
        key_hi = keyn.astype(jnp.bfloat16).astype(jnp.float32)
        tcat_ref[...] = jnp.zeros((NPAD, TPADC), jnp.float32)
        tcat_ref[:NCLS, :KEY_SZ] = anchor_ref[...]
        tcat_ref[:NCLS, KEY_SZ:2 * KEY_SZ] = key_hi
        tcat_ref[:NCLS, 2 * KEY_SZ:3 * KEY_SZ] = keyn - key_hi
        tcat_ref[:NCLS, 3 * KEY_SZ:TCOLS] = aw2
        racc_ref[...] = jnp.zeros((1, KEY_SZ), jnp.float32)

    x = x_ref[...]
    ss = jnp.sum(x * x, axis=1, keepdims=True)
    xn = x * jax.lax.rsqrt(jnp.maximum(ss, 1e-12))

    sim = jax.lax.dot_general(
        xn, keyn_ref[...], (((1,), (1,)), ((), ())),
        precision=_DEF, preferred_element_type=jnp.float32)

    m = jnp.max(sim, axis=1, keepdims=True)
    iota = jax.lax.broadcasted_iota(jnp.int32, (BLK, NCLS), 1)
    idx = jnp.min(jnp.where(sim == m, iota, NCLS), axis=1, keepdims=True)
    iota_p = jax.lax.broadcasted_iota(jnp.int32, (BLK, NPAD), 1)
    onehot = (iota_p == idx).astype(jnp.float32)

    gath = jax.lax.dot_general(
        onehot, tcat_ref[...], (((1,), (0,)), ((), ())),
        precision=_DEF, preferred_element_type=jnp.float32)
    anchor = gath[:, :KEY_SZ]
    keysel = gath[:, KEY_SZ:2 * KEY_SZ] + gath[:, 2 * KEY_SZ:3 * KEY_SZ]
    aw2row = gath[:, 3 * KEY_SZ:TCOLS]

    racc_ref[...] += jnp.sum(keysel * xn, axis=0, keepdims=True)

    out_ref[...] = (jax.lax.dot_general(
                        x, w1_ref[...], (((1,), (0,)), ((), ())),
                        precision=_DEF, preferred_element_type=jnp.float32)
                    + aw2row + b_ref[...])
    x1_ref[:, :KEY_SZ] = x
    x1_ref[:, KEY_SZ:] = anchor

    @pl.when(i == pl.num_programs(0) - 1)
    def _fin():
        rsum_ref[0, 0] = jnp.sum(racc_ref[...])


@jax.jit
def _run(x, key_pool, anchor_pool, w1, w2, b2d):
    grid = BATCH // BLK
    out, x1, rsum = pl.pallas_call(
        _body,
        grid=(grid,),
        in_specs=[
            pl.BlockSpec((BLK, KEY_SZ), lambda i: (i, 0)),
            pl.BlockSpec((NCLS, KEY_SZ), lambda i: (0, 0)),
            pl.BlockSpec((NCLS, KEY_SZ), lambda i: (0, 0)),
            pl.BlockSpec((KEY_SZ, NCLS), lambda i: (0, 0)),
            pl.BlockSpec((KEY_SZ, NCLS), lambda i: (0, 0)),
            pl.BlockSpec((1, NCLS), lambda i: (0, 0)),
        ],
        out_specs=[
            pl.BlockSpec((BLK, NCLS), lambda i: (i, 0)),
            pl.BlockSpec((BLK, 2 * KEY_SZ), lambda i: (i, 0)),
            pl.BlockSpec(memory_space=pltpu.SMEM),
        ],
        out_shape=[
            jax.ShapeDtypeStruct((BATCH, NCLS), jnp.float32),
            jax.ShapeDtypeStruct((BATCH, 2 * KEY_SZ), jnp.float32),
            jax.ShapeDtypeStruct((1, 1), jnp.float32),
        ],
        scratch_shapes=[
            pltpu.VMEM((NCLS, KEY_SZ), jnp.float32),
            pltpu.VMEM((NPAD, TPADC), jnp.float32),
            pltpu.VMEM((1, KEY_SZ), jnp.float32),
        ],
    )(x, key_pool, anchor_pool, w1, w2, b2d)
    return out, x1, rsum[0, 0] / KEY_SZ


def kernel(x, class_mask, key_pool, anchor_pool, W_head, b_head):
    w1 = W_head[:KEY_SZ]
    w2 = W_head[KEY_SZ:]
    b2d = b_head.reshape(1, NCLS)
    return _run(x, key_pool, anchor_pool, w1, w2, b2d)
